# Initial kernel scaffold; baseline (speedup 1.0000x reference)
import jax
import jax.numpy as jnp
from jax.experimental import pallas as pl


def kernel(x, table):
    raise NotImplementedError("write your pallas kernel here")


# speedup vs baseline: 4.7973x; 4.7973x over previous
import jax
import jax.numpy as jnp
from jax import lax
from jax.experimental import pallas as pl
from jax.experimental.pallas import tpu as pltpu
from jax.experimental.pallas import tpu_sc as plsc

NC = 2
NS = 16
NW = NC * NS
ROW = 128
K = 8
CHUNK = K * ROW


def _body(xr, table, out, idx_v, rows_v, sem):
    c = lax.axis_index("c")
    s = lax.axis_index("s")
    wid = s * NC + c
    chunks = xr.shape[0] // (NW * K)
    row0 = wid * chunks * K

    def one_chunk(g, carry):
        base_row = row0 + g * K
        pltpu.sync_copy(xr.at[pl.ds(base_row, K)], idx_v)
        for i in range(K):
            for j in range(ROW // 16):
                sl = (i, pl.ds(j * 16, 16))
                idx_v[sl] = idx_v[sl] - 1
        copies = [
            pltpu.async_copy(
                table.at[idx_v.at[i]], rows_v.at[pl.ds(i * ROW, ROW)], sem
            )
            for i in range(K)
        ]
        for cp in copies:
            cp.wait()
        pltpu.sync_copy(rows_v, out.at[pl.ds(base_row * ROW, K * ROW)])
        return carry

    lax.fori_loop(0, chunks, one_chunk, 0)


def kernel(x, table):
    B, H = x.shape
    V, D = table.shape
    N = B * H
    xr = x.reshape(N // ROW, ROW)
    mesh = plsc.VectorSubcoreMesh(core_axis_name="c", subcore_axis_name="s")
    run = pl.kernel(
        _body,
        out_type=jax.ShapeDtypeStruct((N, D), jnp.float32),
        mesh=mesh,
        scratch_types=[
            pltpu.VMEM((K, ROW), jnp.int32),
            pltpu.VMEM((CHUNK, D), jnp.float32),
            pltpu.SemaphoreType.DMA,
        ],
        compiler_params=pltpu.CompilerParams(use_tc_tiling_on_sc=False),
    )
    out = run(xr, table)
    return out.reshape(B, H, D)


# speedup vs baseline: 5.0417x; 1.0509x over previous
import jax
import jax.numpy as jnp
from jax import lax
from jax.experimental import pallas as pl
from jax.experimental.pallas import tpu as pltpu
from jax.experimental.pallas import tpu_sc as plsc

NC = 2
NS = 16
NW = NC * NS
ROW = 128
K = 8
CHUNK = K * ROW
NBUF = 2


def _body(xr, table, out, idx0, idx1, rows0, rows1, gs0, gs1, ws0, ws1):
    idxs = (idx0, idx1)
    rows = (rows0, rows1)
    gsem = (gs0, gs1)
    wsem = (ws0, ws1)
    c = lax.axis_index("c")
    s = lax.axis_index("s")
    wid = s * NC + c
    chunks = xr.shape[0] // (NW * K)
    row0 = wid * chunks * K

    def load_dec(grow, b):
        pltpu.sync_copy(xr.at[pl.ds(grow, K)], idxs[b])
        for i in range(K):
            for j in range(ROW // 16):
                sl = (i, pl.ds(j * 16, 16))
                idxs[b][sl] = idxs[b][sl] - 1

    def fire_gathers(b):
        for i in range(K):
            pltpu.async_copy(
                table.at[idxs[b].at[i]], rows[b].at[pl.ds(i * ROW, ROW)], gsem[b]
            )

    def drain_gathers(grow, b):
        pltpu.make_async_copy(out.at[pl.ds(grow * ROW, CHUNK)], rows[b], gsem[b]).wait()

    def fire_wb(grow, b):
        pltpu.async_copy(rows[b], out.at[pl.ds(grow * ROW, CHUNK)], wsem[b])

    def wait_wb(grow, b):
        pltpu.make_async_copy(rows[b], out.at[pl.ds(grow * ROW, CHUNK)], wsem[b]).wait()

    for b in range(NBUF):
        load_dec(row0 + b * K, b)
        fire_gathers(b)

    @pl.loop(0, chunks - NBUF, step=NBUF)
    def main(G):
        for b in range(NBUF):
            grow = row0 + (G + b) * K
            drain_gathers(grow, b)
            fire_wb(grow, b)
            load_dec(grow + NBUF * K, b)
            wait_wb(grow, b)
            fire_gathers(b)

    for b in range(NBUF):
        grow = row0 + (chunks - NBUF + b) * K
        drain_gathers(grow, b)
        fire_wb(grow, b)
        wait_wb(grow, b)


def kernel(x, table):
    B, H = x.shape
    V, D = table.shape
    N = B * H
    xr = x.reshape(N // ROW, ROW)
    mesh = plsc.VectorSubcoreMesh(core_axis_name="c", subcore_axis_name="s")
    run = pl.kernel(
        _body,
        out_type=jax.ShapeDtypeStruct((N, D), jnp.float32),
        mesh=mesh,
        scratch_types=[
            pltpu.VMEM((K, ROW), jnp.int32),
            pltpu.VMEM((K, ROW), jnp.int32),
            pltpu.VMEM((CHUNK, D), jnp.float32),
            pltpu.VMEM((CHUNK, D), jnp.float32),
            pltpu.SemaphoreType.DMA,
            pltpu.SemaphoreType.DMA,
            pltpu.SemaphoreType.DMA,
            pltpu.SemaphoreType.DMA,
        ],
        compiler_params=pltpu.CompilerParams(use_tc_tiling_on_sc=False),
    )
    out = run(xr, table)
    return out.reshape(B, H, D)


# speedup vs baseline: 5.0444x; 1.0005x over previous
import jax
import jax.numpy as jnp
from jax import lax
from jax.experimental import pallas as pl
from jax.experimental.pallas import tpu as pltpu
from jax.experimental.pallas import tpu_sc as plsc

NC = 2
NS = 16
NW = NC * NS
ROW = 128
K = 8
CHUNK = K * ROW
NBUF = 2


def _body(xr, table, out, idx0, idx1, rows0, rows1, gs0, gs1, ws0, ws1):
    idxs = (idx0, idx1)
    rows = (rows0, rows1)
    gsem = (gs0, gs1)
    wsem = (ws0, ws1)
    c = lax.axis_index("c")
    s = lax.axis_index("s")
    wid = s * NC + c
    chunks = xr.shape[0] // (NW * CHUNK)
    row0 = wid * chunks * K

    def load_dec(grow, b):
        pltpu.sync_copy(xr.at[pl.ds(grow * ROW, CHUNK)], idxs[b])
        for j in range(CHUNK // 16):
            sl = pl.ds(j * 16, 16)
            idxs[b][sl] = idxs[b][sl] - 1

    def fire_gather(b):
        pltpu.async_copy(table.at[idxs[b]], rows[b], gsem[b])

    def drain_gather(grow, b):
        pltpu.make_async_copy(out.at[pl.ds(grow * ROW, CHUNK)], rows[b], gsem[b]).wait()

    def fire_wb(grow, b):
        pltpu.async_copy(rows[b], out.at[pl.ds(grow * ROW, CHUNK)], wsem[b])

    def wait_wb(grow, b):
        pltpu.make_async_copy(rows[b], out.at[pl.ds(grow * ROW, CHUNK)], wsem[b]).wait()

    for b in range(NBUF):
        load_dec(row0 + b * K, b)
        fire_gather(b)

    @pl.loop(0, chunks - NBUF, step=NBUF)
    def main(G):
        for b in range(NBUF):
            grow = row0 + (G + b) * K
            drain_gather(grow, b)
            fire_wb(grow, b)
            load_dec(grow + NBUF * K, b)
            wait_wb(grow, b)
            fire_gather(b)

    for b in range(NBUF):
        grow = row0 + (chunks - NBUF + b) * K
        drain_gather(grow, b)
        fire_wb(grow, b)
        wait_wb(grow, b)


def kernel(x, table):
    B, H = x.shape
    V, D = table.shape
    N = B * H
    xr = x.reshape(N)
    mesh = plsc.VectorSubcoreMesh(core_axis_name="c", subcore_axis_name="s")
    run = pl.kernel(
        _body,
        out_type=jax.ShapeDtypeStruct((N, D), jnp.float32),
        mesh=mesh,
        scratch_types=[
            pltpu.VMEM((CHUNK,), jnp.int32),
            pltpu.VMEM((CHUNK,), jnp.int32),
            pltpu.VMEM((CHUNK, D), jnp.float32),
            pltpu.VMEM((CHUNK, D), jnp.float32),
            pltpu.SemaphoreType.DMA,
            pltpu.SemaphoreType.DMA,
            pltpu.SemaphoreType.DMA,
            pltpu.SemaphoreType.DMA,
        ],
        compiler_params=pltpu.CompilerParams(use_tc_tiling_on_sc=False),
    )
    out = run(xr, table)
    return out.reshape(B, H, D)
